# baseline (speedup 1.0000x reference)
import jax
import jax.numpy as jnp
from jax.experimental import pallas as pl

_H, _W, _C = 100, 152, 256
_HP, _WP = _H + 2, _W + 2
_P = _HP * _WP
_S = 1968
_NCHUNK = 8
_Q = _S * _NCHUNK
_GUARD = 160
_N = _GUARD + _Q + 160


def _rpn_head(x_ref, w9_ref, wc_ref, b3_ref, bc_ref, out_ref):
    for c in range(_NCHUNK):
        xcat = jnp.concatenate(
            [x_ref[0, :, _GUARD + c * _S + (di - 1) * _WP + (dj - 1):
                   _GUARD + c * _S + (di - 1) * _WP + (dj - 1) + _S]
             for di in range(3) for dj in range(3)], axis=0)
        acc = jnp.dot(w9_ref[...], xcat, preferred_element_type=jnp.float32)
        h = jnp.maximum(acc + b3_ref[...], 0.0).astype(jnp.bfloat16)
        out_ref[0, :, c * _S:(c + 1) * _S] = (
            jnp.dot(wc_ref[...], h, preferred_element_type=jnp.float32)
            + bc_ref[...])


def kernel(x, conv3_w, conv3_b, cls_w, cls_b, reg_w, reg_b):
    n = x.shape[0]
    xp = jnp.pad(x, ((0, 0), (0, 0), (1, 1), (1, 1)))
    xf = jnp.pad(xp.reshape(n, _C, _P),
                 ((0, 0), (0, 0), (_GUARD, _N - _GUARD - _P)))
    xf = xf.astype(jnp.bfloat16)
    w9 = jnp.transpose(conv3_w, (0, 2, 3, 1)).reshape(_C, 9 * _C)
    w9 = w9.astype(jnp.bfloat16)
    wc = jnp.concatenate([reg_w, cls_w], axis=0)[:, :, 0, 0]
    wc = jnp.pad(wc, ((0, 1), (0, 0))).astype(jnp.bfloat16)
    bc = jnp.pad(jnp.concatenate([reg_b, cls_b]), (0, 1)).reshape(16, 1)
    b3 = conv3_b.reshape(_C, 1)

    out = pl.pallas_call(
        _rpn_head,
        grid=(n,),
        in_specs=[
            pl.BlockSpec((1, _C, _N), lambda i: (i, 0, 0)),
            pl.BlockSpec((_C, 9 * _C), lambda i: (0, 0)),
            pl.BlockSpec((16, _C), lambda i: (0, 0)),
            pl.BlockSpec((_C, 1), lambda i: (0, 0)),
            pl.BlockSpec((16, 1), lambda i: (0, 0)),
        ],
        out_specs=pl.BlockSpec((1, 16, _Q), lambda i: (i, 0, 0)),
        out_shape=jax.ShapeDtypeStruct((n, 16, _Q), jnp.float32),
    )(xf, w9, wc, b3, bc)

    o = out[:, :, :_P].reshape(n, 16, _HP, _WP)[:, :, 1:_H + 1, 1:_W + 1]
    o = jnp.transpose(o, (0, 2, 3, 1))
    box = o[..., :12].reshape(n, _H * _W * 3, 4)
    cls = o[..., 12:15].reshape(n, _H * _W * 3, 1)
    return (box, cls)


# speedup vs baseline: 1.6755x; 1.6755x over previous
import jax
import jax.numpy as jnp
from jax.experimental import pallas as pl
from jax.experimental.pallas import tpu as pltpu

_H, _W, _C = 100, 152, 256
_HW = _H * _W
_WP = _W + 2
_RG = 10
_NG = _H // _RG
_SL = _RG * _WP
_G0 = 8
_NP = 15880


def _rpn_head(x_ref, w9_ref, wc_ref, b3_ref, bc_ref, box_ref, cls_ref,
              xp_ref):
    g = pl.program_id(1)

    @pl.when(g == 0)
    def _build_padded():
        xp_ref[...] = jnp.zeros((_C, _NP), jnp.bfloat16)
        for r in range(_H):
            dst = _G0 + (r + 1) * _WP + 1
            xp_ref[:, dst:dst + _W] = (
                x_ref[0, :, r * _W:(r + 1) * _W].astype(jnp.bfloat16))

    def body(gi):
        base = _G0 + (gi * _RG + 1) * _WP
        acc = jnp.zeros((_C, _SL), jnp.float32)
        for k in range(9):
            di, dj = divmod(k, 3)
            start = base + (di - 1) * _WP + (dj - 1)
            acc += jnp.dot(w9_ref[k], xp_ref[:, start:start + _SL],
                           preferred_element_type=jnp.float32)
        h = jnp.maximum(acc + b3_ref[...], 0.0).astype(jnp.bfloat16)
        out16 = (jnp.dot(wc_ref[...], h, preferred_element_type=jnp.float32)
                 + bc_ref[...])
        t = jnp.transpose(out16, (1, 0))
        for rr in range(_RG):
            src = rr * _WP + 1
            row = t[src:src + _W, :]
            box_ref[0, rr * _W:(rr + 1) * _W, :] = row[:, :12]
            cls_ref[0, rr * _W:(rr + 1) * _W, :] = row[:, 12:15]

    for gi in range(_NG):
        pl.when(g == gi)(lambda gi=gi: body(gi))


def kernel(x, conv3_w, conv3_b, cls_w, cls_b, reg_w, reg_b):
    n = x.shape[0]
    xin = x.reshape(n, _C, _HW)
    w9 = jnp.transpose(conv3_w, (2, 3, 0, 1)).reshape(9, _C, _C)
    w9 = w9.astype(jnp.bfloat16)
    wc = jnp.concatenate([reg_w, cls_w], axis=0)[:, :, 0, 0]
    wc = jnp.pad(wc, ((0, 1), (0, 0))).astype(jnp.bfloat16)
    bc = jnp.pad(jnp.concatenate([reg_b, cls_b]), (0, 1)).reshape(16, 1)
    b3 = conv3_b.reshape(_C, 1)

    box, cls = pl.pallas_call(
        _rpn_head,
        grid=(n, _NG),
        in_specs=[
            pl.BlockSpec((1, _C, _HW), lambda i, g: (i, 0, 0)),
            pl.BlockSpec((9, _C, _C), lambda i, g: (0, 0, 0)),
            pl.BlockSpec((16, _C), lambda i, g: (0, 0)),
            pl.BlockSpec((_C, 1), lambda i, g: (0, 0)),
            pl.BlockSpec((16, 1), lambda i, g: (0, 0)),
        ],
        out_specs=[
            pl.BlockSpec((1, _RG * _W, 12), lambda i, g: (i, g, 0)),
            pl.BlockSpec((1, _RG * _W, 3), lambda i, g: (i, g, 0)),
        ],
        out_shape=[
            jax.ShapeDtypeStruct((n, _HW, 12), jnp.float32),
            jax.ShapeDtypeStruct((n, _HW, 3), jnp.float32),
        ],
        scratch_shapes=[pltpu.VMEM((_C, _NP), jnp.bfloat16)],
    )(xin, w9, wc, b3, bc)

    return (box.reshape(n, _HW * 3, 4), cls.reshape(n, _HW * 3, 1))


# speedup vs baseline: 1.7913x; 1.0691x over previous
import jax
import jax.numpy as jnp
from jax.experimental import pallas as pl
from jax.experimental.pallas import tpu as pltpu

_H, _W, _C = 100, 152, 256
_HW = _H * _W
_WP = _W + 2
_RG = 25
_NG = _H // _RG
_SL = _RG * _WP
_G0 = 8
_NP = 15880


def _rpn_head(x_ref, w9_ref, wc_ref, b3_ref, bc_ref, box_ref, cls_ref,
              xp_ref):
    g = pl.program_id(1)

    @pl.when(g == 0)
    def _build_padded():
        xp_ref[...] = jnp.zeros((_C, _NP), jnp.bfloat16)
        for r in range(_H):
            dst = _G0 + (r + 1) * _WP + 1
            xp_ref[:, dst:dst + _W] = (
                x_ref[0, :, r * _W:(r + 1) * _W].astype(jnp.bfloat16))

    def body(gi):
        base = _G0 + (gi * _RG + 1) * _WP
        acc = jnp.zeros((_C, _SL), jnp.float32)
        for k in range(9):
            di, dj = divmod(k, 3)
            start = base + (di - 1) * _WP + (dj - 1)
            acc += jnp.dot(w9_ref[k], xp_ref[:, start:start + _SL],
                           preferred_element_type=jnp.float32)
        h = jnp.maximum(acc + b3_ref[...], 0.0).astype(jnp.bfloat16)
        out16 = (jnp.dot(wc_ref[...], h, preferred_element_type=jnp.float32)
                 + bc_ref[...])
        t = jnp.transpose(out16, (1, 0))
        for rr in range(_RG):
            src = rr * _WP + 1
            row = t[src:src + _W, :]
            box_ref[0, rr * _W:(rr + 1) * _W, :] = row[:, :12]
            cls_ref[0, rr * _W:(rr + 1) * _W, :] = row[:, 12:15]

    for gi in range(_NG):
        pl.when(g == gi)(lambda gi=gi: body(gi))


def kernel(x, conv3_w, conv3_b, cls_w, cls_b, reg_w, reg_b):
    n = x.shape[0]
    xin = x.reshape(n, _C, _HW)
    w9 = jnp.transpose(conv3_w, (2, 3, 0, 1)).reshape(9, _C, _C)
    w9 = w9.astype(jnp.bfloat16)
    wc = jnp.concatenate([reg_w, cls_w], axis=0)[:, :, 0, 0]
    wc = jnp.pad(wc, ((0, 1), (0, 0))).astype(jnp.bfloat16)
    bc = jnp.pad(jnp.concatenate([reg_b, cls_b]), (0, 1)).reshape(16, 1)
    b3 = conv3_b.reshape(_C, 1)

    box, cls = pl.pallas_call(
        _rpn_head,
        grid=(n, _NG),
        in_specs=[
            pl.BlockSpec((1, _C, _HW), lambda i, g: (i, 0, 0)),
            pl.BlockSpec((9, _C, _C), lambda i, g: (0, 0, 0)),
            pl.BlockSpec((16, _C), lambda i, g: (0, 0)),
            pl.BlockSpec((_C, 1), lambda i, g: (0, 0)),
            pl.BlockSpec((16, 1), lambda i, g: (0, 0)),
        ],
        out_specs=[
            pl.BlockSpec((1, _RG * _W, 12), lambda i, g: (i, g, 0)),
            pl.BlockSpec((1, _RG * _W, 3), lambda i, g: (i, g, 0)),
        ],
        out_shape=[
            jax.ShapeDtypeStruct((n, _HW, 12), jnp.float32),
            jax.ShapeDtypeStruct((n, _HW, 3), jnp.float32),
        ],
        scratch_shapes=[pltpu.VMEM((_C, _NP), jnp.bfloat16)],
    )(xin, w9, wc, b3, bc)

    return (box.reshape(n, _HW * 3, 4), cls.reshape(n, _HW * 3, 1))
